# Initial kernel scaffold; baseline (speedup 1.0000x reference)
import jax
import jax.numpy as jnp
from jax.experimental import pallas as pl


def kernel(features, adjs, label, W_s, a_src, a_dst, pos_emb, Wq, Wk, Wv, W_embed, b_embed, W_fc, b_fc):
    raise NotImplementedError("write your pallas kernel here")


# speedup vs baseline: 2.5928x; 2.5928x over previous
import jax
import jax.numpy as jnp
from jax.experimental import pallas as pl
from jax.experimental.pallas import tpu as pltpu

T, N, F = 8, 2048, 128
H_S, D_S = 8, 128
H_T, D_T = 8, 128
DH_S = D_S // H_S
DH_T = D_T // H_T
BN = 256
NB = N // BN


def _leaky(x):
    return jnp.maximum(x, 0.2 * x)


def _structural_kernel(feat_ref, adj_ref, w2_ref, a_ref, out_ref,
                       h_s, ab_s, abt_s):
    i = pl.program_id(1)

    @pl.when(i == 0)
    def _():
        x = feat_ref[0]
        h = jnp.dot(x, w2_ref[...], preferred_element_type=jnp.float32)
        h_s[...] = h
        ab = jnp.dot(h, a_ref[...], preferred_element_type=jnp.float32)
        ab_s[...] = ab
        abt_s[...] = ab.T

    h = h_s[...]
    abt = abt_s[...]
    adj = adj_ref[0]
    mask = (adj > 0.99).astype(jnp.float32)

    h_blk = h_s[pl.ds(i * BN, BN), :]
    ab_blk = ab_s[pl.ds(i * BN, BN), :]
    sub = adj_ref[0, :, pl.ds(i * BN, BN)]
    rows = jax.lax.broadcasted_iota(jnp.int32, (BN, BN), 0)
    cols = jax.lax.broadcasted_iota(jnp.int32, (BN, BN), 1)
    m_d = jnp.sum(jnp.where((rows == cols) & (sub > 0.99), 1.0, 0.0),
                  axis=1)

    outs = []
    for hd in range(H_S):
        b_row = abt[8 + hd:9 + hd, :]
        a_n = ab_blk[:, hd:hd + 1]
        maxb = jnp.max(b_row)
        r = _leaky(a_n + maxb)
        s = a_n + b_row
        p = jnp.exp(_leaky(s) - r) * mask
        hh = h[:, hd * DH_S:(hd + 1) * DH_S]
        num = jnp.dot(p, hh, preferred_element_type=jnp.float32)
        den = jnp.sum(p, axis=1)
        b_diag = ab_blk[:, 8 + hd]
        w_d = jnp.exp(_leaky(a_n[:, 0] + b_diag) - r[:, 0]) * (1.0 - m_d)
        den = den + w_d
        num = num + w_d[:, None] * h_blk[:, hd * DH_S:(hd + 1) * DH_S]
        outs.append(num / den[:, None])
    y = jnp.concatenate(outs, axis=1)
    y = jnp.where(y > 0, y, jnp.exp(jnp.minimum(y, 0.0)) - 1.0)
    out_ref[0] = y + feat_ref[0, pl.ds(i * BN, BN), :]


def _temporal_kernel(sout_ref, pos_ref, wq_ref, wk_ref, wv_ref,
                     s_ref, st_ref, wemb_ref, bemb_ref, wfc_ref, bfc_ref,
                     logist_ref, emb_ref):
    xs = [sout_ref[t] + pos_ref[t:t + 1, :] for t in range(T)]
    q = jnp.dot(xs[T - 1], wq_ref[...], preferred_element_type=jnp.float32)
    scores = []
    vs = []
    for t in range(T):
        k = jnp.dot(xs[t], wk_ref[...], preferred_element_type=jnp.float32)
        vs.append(jnp.dot(xs[t], wv_ref[...],
                          preferred_element_type=jnp.float32))
        scores.append(jnp.dot(q * k, s_ref[...],
                              preferred_element_type=jnp.float32) * 0.25)
    m = scores[0]
    for t in range(1, T):
        m = jnp.maximum(m, scores[t])
    ws = [jnp.exp(sc - m) for sc in scores]
    den = ws[0]
    for t in range(1, T):
        den = den + ws[t]
    out = jnp.zeros_like(vs[0])
    for t in range(T):
        out = out + jnp.dot(ws[t], st_ref[...],
                            preferred_element_type=jnp.float32) * vs[t]
    out = out / jnp.dot(den, st_ref[...], preferred_element_type=jnp.float32)
    emb_ref[...] = out
    embed = jnp.maximum(
        jnp.dot(out, wemb_ref[...], preferred_element_type=jnp.float32)
        + bemb_ref[...], 0.0)
    logist_ref[...] = (jnp.dot(embed, wfc_ref[...],
                               preferred_element_type=jnp.float32)
                       + bfc_ref[...])


def kernel(features, adjs, label, W_s, a_src, a_dst, pos_emb, Wq, Wk, Wv,
           W_embed, b_embed, W_fc, b_fc):
    w2 = jnp.transpose(W_s, (1, 0, 2)).reshape(F, H_S * DH_S)
    idx = jnp.arange(H_S * DH_S)
    amat = jnp.zeros((H_S * DH_S, 16), jnp.float32)
    amat = amat.at[idx, idx // DH_S].set(a_src.reshape(-1))
    amat = amat.at[idx, 8 + idx // DH_S].set(a_dst.reshape(-1))

    sout = pl.pallas_call(
        _structural_kernel,
        grid=(T, NB),
        in_specs=[
            pl.BlockSpec((1, N, F), lambda t, i: (t, 0, 0)),
            pl.BlockSpec((1, BN, N), lambda t, i: (t, i, 0)),
            pl.BlockSpec((F, H_S * DH_S), lambda t, i: (0, 0)),
            pl.BlockSpec((H_S * DH_S, 16), lambda t, i: (0, 0)),
        ],
        out_specs=pl.BlockSpec((1, BN, D_S), lambda t, i: (t, i, 0)),
        out_shape=jax.ShapeDtypeStruct((T, N, D_S), jnp.float32),
        scratch_shapes=[
            pltpu.VMEM((N, H_S * DH_S), jnp.float32),
            pltpu.VMEM((N, 16), jnp.float32),
            pltpu.VMEM((16, N), jnp.float32),
        ],
        compiler_params=pltpu.CompilerParams(
            dimension_semantics=("arbitrary", "arbitrary")),
    )(features, adjs, w2, amat)

    smat = (jnp.arange(D_T)[:, None] // DH_T
            == jnp.arange(H_T)[None, :]).astype(jnp.float32)

    logist, emb = pl.pallas_call(
        _temporal_kernel,
        grid=(NB,),
        in_specs=[
            pl.BlockSpec((T, BN, D_S), lambda i: (0, i, 0)),
            pl.BlockSpec((T, D_S), lambda i: (0, 0)),
            pl.BlockSpec((D_S, D_T), lambda i: (0, 0)),
            pl.BlockSpec((D_S, D_T), lambda i: (0, 0)),
            pl.BlockSpec((D_S, D_T), lambda i: (0, 0)),
            pl.BlockSpec((D_T, H_T), lambda i: (0, 0)),
            pl.BlockSpec((H_T, D_T), lambda i: (0, 0)),
            pl.BlockSpec((D_T, 8), lambda i: (0, 0)),
            pl.BlockSpec((1, 8), lambda i: (0, 0)),
            pl.BlockSpec((8, 1), lambda i: (0, 0)),
            pl.BlockSpec((1, 1), lambda i: (0, 0)),
        ],
        out_specs=[
            pl.BlockSpec((BN, 1), lambda i: (i, 0)),
            pl.BlockSpec((BN, D_T), lambda i: (i, 0)),
        ],
        out_shape=[
            jax.ShapeDtypeStruct((N, 1), jnp.float32),
            jax.ShapeDtypeStruct((N, D_T), jnp.float32),
        ],
    )(sout, pos_emb, Wq, Wk, Wv, smat, smat.T, W_embed,
      b_embed.reshape(1, 8), W_fc, b_fc.reshape(1, 1))

    return logist, emb
